# Initial kernel scaffold; baseline (speedup 1.0000x reference)
import jax
import jax.numpy as jnp
from jax.experimental import pallas as pl


def kernel(x, edge_index, Wl0, Wr0, b0, Wl1, Wr1, b1):
    raise NotImplementedError("write your pallas kernel here")


# speedup vs baseline: 4.7190x; 4.7190x over previous
import functools

import jax
import jax.numpy as jnp
from jax import lax
from jax.experimental import pallas as pl
from jax.experimental.pallas import tpu as pltpu
from jax.experimental.pallas import tpu_sc as plsc

N = 10000
E = 320000
D = 128

_INFO = plsc.get_sparse_core_info()
NC = _INFO.num_cores
NS = _INFO.num_subcores
NW = NC * NS
EPW = E // NW
K = 80
NCHUNK = EPW // K
RPT = N // NS
ZROWS = 125
WPT = 640
WTAIL0 = (NS - 1) * WPT
NUP = NS * WPT

_MESH = plsc.VectorSubcoreMesh(core_axis_name="c", subcore_axis_name="s")


@functools.partial(
    pl.kernel, mesh=_MESH,
    out_type=jax.ShapeDtypeStruct((NC, N, D), jnp.float32),
    scratch_types=[
        pltpu.VMEM_SHARED((N, D), jnp.float32),
        pltpu.VMEM((ZROWS, D), jnp.float32),
        pltpu.VMEM((K,), jnp.int32),
        pltpu.VMEM((K,), jnp.int32),
        pltpu.VMEM((K, D), jnp.float32),
        pltpu.SemaphoreType.DMA,
    ])
def _sc_agg(h_hbm, src_hbm, dst_hbm, zf_hbm, agg_out,
            acc, zbuf, sidx, didx, rows, sem):
    c = lax.axis_index("c")
    s = lax.axis_index("s")
    wid = s * NC + c

    pltpu.sync_copy(zf_hbm, zbuf)
    for z in range(RPT // ZROWS):
        pltpu.sync_copy(zbuf, acc.at[pl.ds(s * RPT + z * ZROWS, ZROWS)])
    plsc.subcore_barrier()

    def body(i, carry):
        base = pl.multiple_of(wid * EPW + i * K, 8)
        pltpu.sync_copy(src_hbm.at[pl.ds(base, K)], sidx)
        pltpu.sync_copy(dst_hbm.at[pl.ds(base, K)], didx)
        pltpu.async_copy(h_hbm.at[sidx], rows, sem).wait()
        pltpu.sync_copy(rows, acc.at[didx], add=True)
        return carry

    lax.fori_loop(0, NCHUNK, body, 0)
    plsc.subcore_barrier()

    w0 = pl.multiple_of(s * WPT, 8)

    @pl.when(s < NS - 1)
    def _write_full():
        pltpu.sync_copy(acc.at[pl.ds(w0, WPT)], agg_out.at[c, pl.ds(w0, WPT)])

    @pl.when(s == NS - 1)
    def _write_tail():
        pltpu.sync_copy(acc.at[pl.ds(WTAIL0, N - WTAIL0)],
                        agg_out.at[c, pl.ds(WTAIL0, N - WTAIL0)])


@functools.partial(
    pl.kernel, mesh=_MESH,
    out_type=jax.ShapeDtypeStruct((NC, NUP), jnp.float32),
    scratch_types=[
        pltpu.VMEM_SHARED((NUP,), jnp.float32),
        pltpu.VMEM((WPT,), jnp.float32),
        pltpu.VMEM((K,), jnp.int32),
        pltpu.VMEM((K,), jnp.float32),
    ])
def _sc_deg(dst_hbm, zd_hbm, ones_hbm, deg_out, dacc, zdbuf, didx, ones_v):
    c = lax.axis_index("c")
    s = lax.axis_index("s")
    wid = s * NC + c
    w0 = pl.multiple_of(s * WPT, 128)

    pltpu.sync_copy(zd_hbm, zdbuf)
    pltpu.sync_copy(ones_hbm, ones_v)
    pltpu.sync_copy(zdbuf, dacc.at[pl.ds(w0, WPT)])
    plsc.subcore_barrier()

    def body(i, carry):
        base = pl.multiple_of(wid * EPW + i * K, 8)
        pltpu.sync_copy(dst_hbm.at[pl.ds(base, K)], didx)
        pltpu.sync_copy(ones_v, dacc.at[didx], add=True)
        return carry

    lax.fori_loop(0, NCHUNK, body, 0)
    plsc.subcore_barrier()

    pltpu.sync_copy(dacc.at[pl.ds(w0, WPT)], deg_out.at[c, pl.ds(w0, WPT)])


def _tc_layer_body(relu, p_ref, d_ref, h_ref, wl_ref, wr_ref, b_ref, o_ref):
    agg = p_ref[0] + p_ref[1]
    deg = d_ref[...]
    mean = agg / jnp.maximum(deg, 1.0)
    out = (jnp.dot(mean, wl_ref[...], preferred_element_type=jnp.float32)
           + jnp.dot(h_ref[...], wr_ref[...], preferred_element_type=jnp.float32)
           + b_ref[...])
    if relu:
        out = jnp.maximum(out, 0.0)
    o_ref[...] = out


def _tc_layer(aggp, deg_col, h, Wl, Wr, b, relu):
    BN = 1000
    grid = (N // BN,)
    return pl.pallas_call(
        functools.partial(_tc_layer_body, relu),
        grid=grid,
        in_specs=[
            pl.BlockSpec((NC, BN, D), lambda i: (0, i, 0)),
            pl.BlockSpec((BN, 1), lambda i: (i, 0)),
            pl.BlockSpec((BN, D), lambda i: (i, 0)),
            pl.BlockSpec((D, D), lambda i: (0, 0)),
            pl.BlockSpec((D, D), lambda i: (0, 0)),
            pl.BlockSpec((1, D), lambda i: (0, 0)),
        ],
        out_specs=pl.BlockSpec((BN, D), lambda i: (i, 0)),
        out_shape=jax.ShapeDtypeStruct((N, D), jnp.float32),
    )(aggp, deg_col, h, Wl, Wr, b.reshape(1, D))


def kernel(x, edge_index, Wl0, Wr0, b0, Wl1, Wr1, b1):
    src = edge_index[0]
    dst = edge_index[1]
    zf = jnp.zeros((ZROWS, D), jnp.float32)
    zd = jnp.zeros((WPT,), jnp.float32)
    ones = jnp.ones((K,), jnp.float32)

    degp = _sc_deg(dst, zd, ones)
    deg_col = (degp[0, :N] + degp[1, :N]).reshape(N, 1)
    aggp0 = _sc_agg(x, src, dst, zf)
    h1 = _tc_layer(aggp0, deg_col, x, Wl0, Wr0, b0, relu=True)
    aggp1 = _sc_agg(h1, src, dst, zf)
    out = _tc_layer(aggp1, deg_col, h1, Wl1, Wr1, b1, relu=False)
    return out


# speedup vs baseline: 9.3581x; 1.9831x over previous
import functools

import jax
import jax.numpy as jnp
from jax import lax
from jax.experimental import pallas as pl
from jax.experimental.pallas import tpu as pltpu
from jax.experimental.pallas import tpu_sc as plsc

N = 10000
E = 320000
D = 128

_INFO = plsc.get_sparse_core_info()
NC = _INFO.num_cores
NS = _INFO.num_subcores
NW = NC * NS
EPW = E // NW
K = 80
NCHUNK = EPW // K
RPT = N // NS
ZROWS = 125
WPT = 640
WTAIL0 = (NS - 1) * WPT
NUP = NS * WPT

_MESH = plsc.VectorSubcoreMesh(core_axis_name="c", subcore_axis_name="s")


def _make_sc_agg(with_deg):
    out_type = [jax.ShapeDtypeStruct((NC, N, D), jnp.float32)]
    scratch = [
        pltpu.VMEM_SHARED((N, D), jnp.float32),
        pltpu.VMEM((ZROWS, D), jnp.float32),
    ]
    for _ in range(2):
        scratch += [
            pltpu.VMEM((K,), jnp.int32),
            pltpu.VMEM((K,), jnp.int32),
            pltpu.VMEM((K, D), jnp.float32),
            pltpu.SemaphoreType.DMA,
            pltpu.SemaphoreType.DMA,
            pltpu.SemaphoreType.DMA,
        ]
    if with_deg:
        out_type.append(jax.ShapeDtypeStruct((NC, NUP), jnp.float32))
        scratch += [
            pltpu.VMEM_SHARED((NUP,), jnp.float32),
            pltpu.VMEM((WPT,), jnp.float32),
            pltpu.VMEM((K,), jnp.float32),
        ]

    @functools.partial(pl.kernel, mesh=_MESH, out_type=out_type,
                       scratch_types=scratch)
    def k(h_hbm, src_hbm, dst_hbm, zf_hbm, zd_hbm, ones_hbm, *rest):
        if with_deg:
            (agg_out, deg_out, acc, zbuf,
             sidx0, didx0, rows0, ss0, ds0, gs0,
             sidx1, didx1, rows1, ss1, ds1, gs1,
             dacc, zdbuf, ones_v) = rest
        else:
            (agg_out, acc, zbuf,
             sidx0, didx0, rows0, ss0, ds0, gs0,
             sidx1, didx1, rows1, ss1, ds1, gs1) = rest
        sidx = (sidx0, sidx1)
        didx = (didx0, didx1)
        rows = (rows0, rows1)
        ss = (ss0, ss1)
        ds = (ds0, ds1)
        gs = (gs0, gs1)

        c = lax.axis_index("c")
        s = lax.axis_index("s")
        wid = s * NC + c
        w0 = pl.multiple_of(s * WPT, 128)

        pltpu.sync_copy(zf_hbm, zbuf)
        if with_deg:
            pltpu.sync_copy(zd_hbm, zdbuf)
            pltpu.sync_copy(ones_hbm, ones_v)
            pltpu.sync_copy(zdbuf, dacc.at[pl.ds(w0, WPT)])
        for z in range(RPT // ZROWS):
            pltpu.sync_copy(zbuf, acc.at[pl.ds(s * RPT + z * ZROWS, ZROWS)])
        plsc.subcore_barrier()

        def start_idx(ci, b):
            base = pl.multiple_of(wid * EPW + ci * K, 8)
            pltpu.make_async_copy(src_hbm.at[pl.ds(base, K)], sidx[b],
                                  ss[b]).start()
            pltpu.make_async_copy(dst_hbm.at[pl.ds(base, K)], didx[b],
                                  ds[b]).start()

        def wait_idx_start_gather(b):
            pltpu.make_async_copy(src_hbm.at[pl.ds(0, K)], sidx[b],
                                  ss[b]).wait()
            pltpu.make_async_copy(h_hbm.at[sidx[b]], rows[b], gs[b]).start()

        start_idx(0, 0)
        start_idx(1, 1)
        wait_idx_start_gather(0)

        def body(i2, carry):
            for b in (0, 1):
                ci = 2 * i2 + b

                @pl.when(ci < NCHUNK)
                def _():
                    pltpu.make_async_copy(h_hbm.at[sidx[b]], rows[b],
                                          gs[b]).wait()

                    @pl.when(ci + 1 < NCHUNK)
                    def _():
                        wait_idx_start_gather(b ^ 1)

                    pltpu.make_async_copy(dst_hbm.at[pl.ds(0, K)], didx[b],
                                          ds[b]).wait()
                    pltpu.sync_copy(rows[b], acc.at[didx[b]], add=True)
                    if with_deg:
                        pltpu.sync_copy(ones_v, dacc.at[didx[b]], add=True)

                    @pl.when(ci + 2 < NCHUNK)
                    def _():
                        start_idx(ci + 2, b)

            return carry

        lax.fori_loop(0, (NCHUNK + 2) // 2, body, 0)
        plsc.subcore_barrier()

        if with_deg:
            pltpu.sync_copy(dacc.at[pl.ds(w0, WPT)],
                            deg_out.at[c, pl.ds(w0, WPT)])

        @pl.when(s < NS - 1)
        def _write_full():
            pltpu.sync_copy(acc.at[pl.ds(w0, WPT)],
                            agg_out.at[c, pl.ds(w0, WPT)])

        @pl.when(s == NS - 1)
        def _write_tail():
            pltpu.sync_copy(acc.at[pl.ds(WTAIL0, N - WTAIL0)],
                            agg_out.at[c, pl.ds(WTAIL0, N - WTAIL0)])

    return k


_sc_agg_deg = _make_sc_agg(True)
_sc_agg_only = _make_sc_agg(False)


def _tc_layer_body(relu, p_ref, d_ref, h_ref, wl_ref, wr_ref, b_ref, o_ref):
    agg = p_ref[0] + p_ref[1]
    deg = d_ref[...]
    mean = agg / jnp.maximum(deg, 1.0)
    out = (jnp.dot(mean, wl_ref[...], preferred_element_type=jnp.float32)
           + jnp.dot(h_ref[...], wr_ref[...], preferred_element_type=jnp.float32)
           + b_ref[...])
    if relu:
        out = jnp.maximum(out, 0.0)
    o_ref[...] = out


def _tc_layer(aggp, deg_col, h, Wl, Wr, b, relu):
    BN = 1000
    grid = (N // BN,)
    return pl.pallas_call(
        functools.partial(_tc_layer_body, relu),
        grid=grid,
        in_specs=[
            pl.BlockSpec((NC, BN, D), lambda i: (0, i, 0)),
            pl.BlockSpec((BN, 1), lambda i: (i, 0)),
            pl.BlockSpec((BN, D), lambda i: (i, 0)),
            pl.BlockSpec((D, D), lambda i: (0, 0)),
            pl.BlockSpec((D, D), lambda i: (0, 0)),
            pl.BlockSpec((1, D), lambda i: (0, 0)),
        ],
        out_specs=pl.BlockSpec((BN, D), lambda i: (i, 0)),
        out_shape=jax.ShapeDtypeStruct((N, D), jnp.float32),
    )(aggp, deg_col, h, Wl, Wr, b.reshape(1, D))


def kernel(x, edge_index, Wl0, Wr0, b0, Wl1, Wr1, b1):
    src = edge_index[0]
    dst = edge_index[1]
    zf = jnp.zeros((ZROWS, D), jnp.float32)
    zd = jnp.zeros((WPT,), jnp.float32)
    ones = jnp.ones((K,), jnp.float32)

    aggp0, degp = _sc_agg_deg(x, src, dst, zf, zd, ones)
    deg_col = (degp[0, :N] + degp[1, :N]).reshape(N, 1)
    h1 = _tc_layer(aggp0, deg_col, x, Wl0, Wr0, b0, relu=True)
    (aggp1,) = _sc_agg_only(h1, src, dst, zf, zd, ones)
    out = _tc_layer(aggp1, deg_col, h1, Wl1, Wr1, b1, relu=False)
    return out
